# baseline (speedup 1.0000x reference)
import jax
import jax.numpy as jnp
from jax import lax
from jax.experimental import pallas as pl
from jax.experimental.pallas import tpu as pltpu
from jax.experimental.pallas import tpu_sc as plsc

_B = 4096
_CV = 72
_NW = 32
_BW = _B // _NW
_NP = _CV * 3
_CT = _CV // 8


def _sc_body(pos_hbm, col_hbm, out_hbm, cpos, ccol, m, pbuf, sem):
    cid = lax.axis_index("c")
    sid = lax.axis_index("s")
    w = sid * 2 + cid
    base = w * _BW
    pltpu.sync_copy(pos_hbm.at[:, pl.ds(base, _BW)], cpos)
    pltpu.sync_copy(col_hbm.at[:, pl.ds(base, _BW)], ccol)
    f255 = jnp.full((16,), 255.0, jnp.float32)

    def mbody(c, carry):
        for k in range(8):
            t1 = cpos[1, pl.ds(k * 16, 16)]
            b1 = cpos[3, pl.ds(k * 16, 16)]
            cv = jnp.full((16,), c, jnp.int32)
            cm = (cv >= t1) & (cv < b1)
            for ch in range(3):
                m[ch, c, pl.ds(k * 16, 16)] = jnp.where(
                    cm, ccol[ch, pl.ds(k * 16, 16)], f255)
        return carry

    lax.fori_loop(0, _CV, mbody, 0)

    def rbody(r, carry):
        par = r & 1

        @pl.when(r >= 2)
        def _wait():
            for ch in range(3):
                pltpu.make_async_copy(
                    pbuf.at[par, ch],
                    out_hbm.at[(r - 2) * 3 + ch, :, w],
                    sem.at[par, ch]).wait()

        for k in range(8):
            t0 = cpos[0, pl.ds(k * 16, 16)]
            b0 = cpos[2, pl.ds(k * 16, 16)]
            rv = jnp.full((16,), r, jnp.int32)
            rm = (rv >= t0) & (rv < b0)
            for ch in range(3):
                def ctbody(ct, c2, _ch=ch, _k=k, _rm=rm, _par=par):
                    for cs in range(8):
                        val = jnp.where(
                            _rm, m[_ch, ct * 8 + cs, pl.ds(_k * 16, 16)], f255)
                        pbuf[_par, _ch, ct, cs, pl.ds(_k * 16, 16)] = val
                    return c2

                lax.fori_loop(0, _CT, ctbody, 0)

        for ch in range(3):
            pltpu.async_copy(
                pbuf.at[par, ch], out_hbm.at[r * 3 + ch, :, w],
                sem.at[par, ch])
        return carry

    lax.fori_loop(0, _CV, rbody, 0)

    for ch in range(3):
        pltpu.make_async_copy(
            pbuf.at[0, ch], out_hbm.at[70 * 3 + ch, :, w],
            sem.at[0, ch]).wait()
        pltpu.make_async_copy(
            pbuf.at[1, ch], out_hbm.at[71 * 3 + ch, :, w],
            sem.at[1, ch]).wait()


def kernel(positions, colors):
    posr = positions.T
    colr = colors.T
    mesh = plsc.VectorSubcoreMesh(core_axis_name="c", subcore_axis_name="s")
    y6 = pl.kernel(
        _sc_body,
        out_type=jax.ShapeDtypeStruct((_NP, _CT, _NW, 8, 128), jnp.float32),
        mesh=mesh,
        scratch_types=[
            pltpu.VMEM((4, _BW), jnp.int32),
            pltpu.VMEM((3, _BW), jnp.float32),
            pltpu.VMEM((3, _CV, _BW), jnp.float32),
            pltpu.VMEM((2, 3, _CT, 8, 128), jnp.float32),
            pltpu.SemaphoreType.DMA((2, 3)),
        ],
    )(posr, colr)
    a = y6.reshape(_CV, 3, _CT, _NW, 8, 128)
    b = a.transpose(3, 5, 0, 2, 4, 1)
    return b.reshape(_B, _CV, _CV, 3)


# speedup vs baseline: 1.8527x; 1.8527x over previous
import jax
import jax.numpy as jnp
from jax import lax
from jax.experimental import pallas as pl
from jax.experimental.pallas import tpu as pltpu
from jax.experimental.pallas import tpu_sc as plsc

_B = 4096
_CV = 72
_NW = 32
_BW = _B // _NW
_NP = _CV * 3
_CT = _CV // 8


def _sc_body(pos_hbm, col_hbm, out_hbm, cpos, ccol, m, pbuf, sem):
    cid = lax.axis_index("c")
    sid = lax.axis_index("s")
    w = sid * 2 + cid
    base = w * _BW
    pltpu.sync_copy(pos_hbm.at[:, pl.ds(base, _BW)], cpos)
    pltpu.sync_copy(col_hbm.at[:, pl.ds(base, _BW)], ccol)
    f255 = jnp.full((16,), 255.0, jnp.float32)

    def mbody(c, carry):
        for k in range(8):
            t1 = cpos[1, pl.ds(k * 16, 16)]
            b1 = cpos[3, pl.ds(k * 16, 16)]
            cv = jnp.full((16,), c, jnp.int32)
            cm = (cv >= t1) & (cv < b1)
            for ch in range(3):
                m[ch, c, pl.ds(k * 16, 16)] = jnp.where(
                    cm, ccol[ch, pl.ds(k * 16, 16)], f255)
        return carry

    lax.fori_loop(0, _CV, mbody, 0)

    def rbody(r, carry):
        par = r & 1

        @pl.when(r >= 2)
        def _wait():
            for ch in range(3):
                pltpu.make_async_copy(
                    pbuf.at[par, ch],
                    out_hbm.at[(r - 2) * 3 + ch, :, w],
                    sem.at[par, ch]).wait()

        for k in range(8):
            t0 = cpos[0, pl.ds(k * 16, 16)]
            b0 = cpos[2, pl.ds(k * 16, 16)]
            rv = jnp.full((16,), r, jnp.int32)
            rm = (rv >= t0) & (rv < b0)
            for ch in range(3):
                @plsc.parallel_loop(0, _CT, 1, unroll=3)
                def _ctbody(ct, _ch=ch, _k=k, _rm=rm, _par=par):
                    for cs in range(8):
                        val = jnp.where(
                            _rm, m[_ch, ct * 8 + cs, pl.ds(_k * 16, 16)], f255)
                        pbuf[_par, _ch, ct, cs, pl.ds(_k * 16, 16)] = val

        for ch in range(3):
            pltpu.async_copy(
                pbuf.at[par, ch], out_hbm.at[r * 3 + ch, :, w],
                sem.at[par, ch])
        return carry

    lax.fori_loop(0, _CV, rbody, 0)

    for ch in range(3):
        pltpu.make_async_copy(
            pbuf.at[0, ch], out_hbm.at[70 * 3 + ch, :, w],
            sem.at[0, ch]).wait()
        pltpu.make_async_copy(
            pbuf.at[1, ch], out_hbm.at[71 * 3 + ch, :, w],
            sem.at[1, ch]).wait()


def kernel(positions, colors):
    posr = positions.T
    colr = colors.T
    mesh = plsc.VectorSubcoreMesh(core_axis_name="c", subcore_axis_name="s")
    y6 = pl.kernel(
        _sc_body,
        out_type=jax.ShapeDtypeStruct((_NP, _CT, _NW, 8, 128), jnp.float32),
        mesh=mesh,
        scratch_types=[
            pltpu.VMEM((4, _BW), jnp.int32),
            pltpu.VMEM((3, _BW), jnp.float32),
            pltpu.VMEM((3, _CV, _BW), jnp.float32),
            pltpu.VMEM((2, 3, _CT, 8, 128), jnp.float32),
            pltpu.SemaphoreType.DMA((2, 3)),
        ],
    )(posr, colr)
    a = y6.reshape(_CV, 3, _CT, _NW, 8, 128)
    b = a.transpose(3, 5, 0, 2, 4, 1)
    return b.reshape(_B, _CV, _CV, 3)


# speedup vs baseline: 1.9141x; 1.0331x over previous
import jax
import jax.numpy as jnp
from jax import lax
from jax.experimental import pallas as pl
from jax.experimental.pallas import tpu as pltpu
from jax.experimental.pallas import tpu_sc as plsc

_B = 4096
_CV = 72
_NW = 32
_BW = _B // _NW
_NP = _CV * 3
_CT = _CV // 8


def _sc_body(pos_hbm, col_hbm, out_hbm, cpos, ccol, m, pbuf, sem):
    cid = lax.axis_index("c")
    sid = lax.axis_index("s")
    w = sid * 2 + cid
    base = w * _BW
    pltpu.sync_copy(pos_hbm.at[:, pl.ds(base, _BW)], cpos)
    pltpu.sync_copy(col_hbm.at[:, pl.ds(base, _BW)], ccol)
    f255 = jnp.full((16,), 255.0, jnp.float32)

    def mbody(c, carry):
        for k in range(8):
            t1 = cpos[1, pl.ds(k * 16, 16)]
            b1 = cpos[3, pl.ds(k * 16, 16)]
            cv = jnp.full((16,), c, jnp.int32)
            cm = (cv >= t1) & (cv < b1)
            for ch in range(3):
                m[ch, c, pl.ds(k * 16, 16)] = jnp.where(
                    cm, ccol[ch, pl.ds(k * 16, 16)], f255)
        return carry

    lax.fori_loop(0, _CV, mbody, 0)

    def rbody(r, carry):
        par = r & 1

        @pl.when(r >= 2)
        def _wait():
            for ch in range(3):
                pltpu.make_async_copy(
                    pbuf.at[par, ch],
                    out_hbm.at[(r - 2) * 3 + ch, :, w],
                    sem.at[par, ch]).wait()

        for k in range(8):
            t0 = cpos[0, pl.ds(k * 16, 16)]
            b0 = cpos[2, pl.ds(k * 16, 16)]
            rv = jnp.full((16,), r, jnp.int32)
            rm = (rv >= t0) & (rv < b0)
            for ch in range(3):
                @plsc.parallel_loop(0, _CT, 1, unroll=9)
                def _ctbody(ct, _ch=ch, _k=k, _rm=rm, _par=par):
                    for cs in range(8):
                        val = jnp.where(
                            _rm, m[_ch, ct * 8 + cs, pl.ds(_k * 16, 16)], f255)
                        pbuf[_par, _ch, ct, cs, pl.ds(_k * 16, 16)] = val

        for ch in range(3):
            pltpu.async_copy(
                pbuf.at[par, ch], out_hbm.at[r * 3 + ch, :, w],
                sem.at[par, ch])
        return carry

    lax.fori_loop(0, _CV, rbody, 0)

    for ch in range(3):
        pltpu.make_async_copy(
            pbuf.at[0, ch], out_hbm.at[70 * 3 + ch, :, w],
            sem.at[0, ch]).wait()
        pltpu.make_async_copy(
            pbuf.at[1, ch], out_hbm.at[71 * 3 + ch, :, w],
            sem.at[1, ch]).wait()


def kernel(positions, colors):
    posr = positions.T
    colr = colors.T
    mesh = plsc.VectorSubcoreMesh(core_axis_name="c", subcore_axis_name="s")
    y6 = pl.kernel(
        _sc_body,
        out_type=jax.ShapeDtypeStruct((_NP, _CT, _NW, 8, 128), jnp.float32),
        mesh=mesh,
        scratch_types=[
            pltpu.VMEM((4, _BW), jnp.int32),
            pltpu.VMEM((3, _BW), jnp.float32),
            pltpu.VMEM((3, _CV, _BW), jnp.float32),
            pltpu.VMEM((2, 3, _CT, 8, 128), jnp.float32),
            pltpu.SemaphoreType.DMA((2, 3)),
        ],
    )(posr, colr)
    a = y6.reshape(_CV, 3, _CT, _NW, 8, 128)
    b = a.transpose(3, 5, 0, 2, 4, 1)
    return b.reshape(_B, _CV, _CV, 3)


# speedup vs baseline: 6.3845x; 3.3355x over previous
import jax
import jax.numpy as jnp
from jax import lax
from jax.experimental import pallas as pl

_B = 4096
_CV = 72
_RB = 8
_BB = 4096


def _tc_body(pos_ref, col_ref, out_ref):
    i = pl.program_id(0)
    t0 = pos_ref[0:1]
    t1 = pos_ref[1:2]
    b0 = pos_ref[2:3]
    b1 = pos_ref[3:4]
    riota = lax.broadcasted_iota(jnp.int32, (_RB, 1, 1, _BB), 0) + i * _RB
    ciota = lax.broadcasted_iota(jnp.int32, (1, 1, _CV, _BB), 2)
    rowm = (riota >= t0) & (riota < b0)
    colm = (ciota >= t1) & (ciota < b1)
    mask = rowm & colm
    colv = col_ref[...].reshape(1, 3, 1, _BB)
    out_ref[...] = jnp.where(mask, colv, jnp.float32(255.0))


def kernel(positions, colors):
    posr = positions.T.reshape(4, 1, 1, _B)
    colr = colors.T.reshape(3, 1, 1, _B)
    y = pl.pallas_call(
        _tc_body,
        grid=(_CV // _RB, _B // _BB),
        in_specs=[
            pl.BlockSpec((4, 1, 1, _BB), lambda i, j: (0, 0, 0, j)),
            pl.BlockSpec((3, 1, 1, _BB), lambda i, j: (0, 0, 0, j)),
        ],
        out_specs=pl.BlockSpec((_RB, 3, _CV, _BB), lambda i, j: (i, 0, 0, j)),
        out_shape=jax.ShapeDtypeStruct((_CV, 3, _CV, _B), jnp.float32),
    )(posr, colr)
    return jnp.transpose(y, (3, 0, 2, 1))
